# baseline (speedup 1.0000x reference)
import functools

import jax
import jax.numpy as jnp
from jax import lax
from jax.experimental import pallas as pl
from jax.experimental.pallas import tpu as pltpu
from jax.experimental.pallas import tpu_sc as plsc

_NC = 2
_NS = 16
_NW = _NC * _NS
_G = 128
_D = 64
_LANES = 16

_V = 1000000
_COLS = _V // 128
_COLS_PER_W = _COLS // _NW
_CHUNK_COLS = 2
_W = _CHUNK_COLS * 128
_TAIL_FULL = _COLS - _COLS_PER_W * _NW
_TAIL_LANES = _V - _COLS * 128


def _build_transpose():
    mesh = plsc.VectorSubcoreMesh(core_axis_name="c", subcore_axis_name="s")

    @functools.partial(
        pl.kernel,
        mesh=mesh,
        compiler_params=pltpu.CompilerParams(needs_layout_passes=False),
        out_type=jax.ShapeDtypeStruct((_V * _D,), jnp.float32),
        scratch_types=[
            pltpu.VMEM((_D, _W), jnp.float32),
            pltpu.VMEM((_W * _D,), jnp.float32),
            pltpu.VMEM((_D, _TAIL_LANES), jnp.float32),
            pltpu.VMEM((_TAIL_LANES * _D,), jnp.float32),
        ],
    )
    def body(wt_hbm, t2_hbm, src_v, dst_v, tsrc_v, tdst_v):
        wid = lax.axis_index("s") * _NC + lax.axis_index("c")
        iota = lax.iota(jnp.int32, _LANES)
        lane0 = wid * (_COLS_PER_W * 128)

        def transpose_block(src, dst, width):
            def jgroup(j, carry):
                l0 = j * _LANES
                dbase = (l0 + iota) * _D
                def dstep(d8, c2):
                    for dd in range(8):
                        d = d8 * 8 + dd
                        vals = src[d, pl.ds(l0, _LANES)]
                        plsc.store_scatter(dst, [dbase + d], vals)
                    return c2
                lax.fori_loop(0, _D // 8, dstep, 0)
                return carry
            lax.fori_loop(0, width // _LANES, jgroup, 0)

        def chunk(i, carry):
            lane = lane0 + i * _W
            pltpu.sync_copy(wt_hbm.at[:, pl.ds(lane, _W)], src_v)
            transpose_block(src_v, dst_v, _W)
            pltpu.sync_copy(dst_v, t2_hbm.at[pl.ds(lane * _D, _W * _D)])
            return carry

        lax.fori_loop(0, _COLS_PER_W * 128 // _W, chunk, 0)

        @pl.when(wid < _TAIL_FULL)
        def _():
            lane = (_COLS_PER_W * _NW + wid) * 128
            pltpu.sync_copy(wt_hbm.at[:, pl.ds(lane, 128)],
                            src_v.at[:, pl.ds(0, 128)])
            def jgroup(j, carry):
                l0 = j * _LANES
                dbase = (l0 + iota) * _D
                def dstep(d8, c2):
                    for dd in range(8):
                        d = d8 * 8 + dd
                        vals = src_v[d, pl.ds(l0, _LANES)]
                        plsc.store_scatter(dst_v, [dbase + d], vals)
                    return c2
                lax.fori_loop(0, _D // 8, dstep, 0)
                return carry
            lax.fori_loop(0, 128 // _LANES, jgroup, 0)
            pltpu.sync_copy(dst_v.at[pl.ds(0, 128 * _D)],
                            t2_hbm.at[pl.ds(lane * _D, 128 * _D)])

        @pl.when(wid == _TAIL_FULL)
        def _():
            lane = _COLS * 128
            pltpu.sync_copy(wt_hbm.at[:, pl.ds(lane, _TAIL_LANES)], tsrc_v)
            def jgroup(j, carry):
                l0 = j * _LANES
                dbase = (l0 + iota) * _D
                def dstep(d8, c2):
                    for dd in range(8):
                        d = d8 * 8 + dd
                        vals = tsrc_v[d, pl.ds(l0, _LANES)]
                        plsc.store_scatter(tdst_v, [dbase + d], vals)
                    return c2
                lax.fori_loop(0, _D // 8, dstep, 0)
                return carry
            lax.fori_loop(0, _TAIL_LANES // _LANES, jgroup, 0)
            pltpu.sync_copy(tdst_v, t2_hbm.at[pl.ds(lane * _D, _TAIL_LANES * _D)])

    return body


def _build_gather(num_granules, total_tokens):
    mesh = plsc.VectorSubcoreMesh(core_axis_name="c", subcore_axis_name="s")

    @functools.partial(
        pl.kernel,
        mesh=mesh,
        compiler_params=pltpu.CompilerParams(needs_layout_passes=False),
        out_type=jax.ShapeDtypeStruct((total_tokens, _D), jnp.float32),
        scratch_types=[
            pltpu.VMEM((num_granules, _G), jnp.int32),
            pltpu.VMEM((_G,), jnp.int32),
            pltpu.VMEM((_G, 128), jnp.float32),
            pltpu.VMEM((_G // 8, 8, _D), jnp.float32),
            pltpu.SemaphoreType.DMA,
        ],
    )
    def body(ids_hbm, t2_hbm, out_hbm, idx_v, pair_v, rows_v, sel_v, sem):
        wid = lax.axis_index("s") * _NC + lax.axis_index("c")
        iota = lax.iota(jnp.int32, _LANES)
        pltpu.sync_copy(ids_hbm.at[wid], idx_v)
        out3 = out_hbm.reshape(total_tokens // 8, 8, _D)

        def step(g, carry):
            def mk_pair(j, c2):
                ids16 = idx_v[g, pl.ds(j * _LANES, _LANES)]
                pair_v[pl.ds(j * _LANES, _LANES)] = lax.shift_right_logical(
                    ids16, 1)
                return c2
            lax.fori_loop(0, _G // _LANES, mk_pair, 0)

            pltpu.async_copy(t2_hbm.at[pair_v], rows_v, sem).wait()

            def sel_group(j, c2):
                r16 = j * _LANES + iota
                ids16 = idx_v[g, pl.ds(j * _LANES, _LANES)]
                colbase = (ids16 & 1) * _D
                q16 = lax.shift_right_logical(r16, 3)
                s16 = r16 & 7
                def dstep(d8, c3):
                    for dd in range(8):
                        d = d8 * 8 + dd
                        vals = plsc.load_gather(rows_v, [r16, colbase + d])
                        plsc.store_scatter(
                            sel_v, [q16, s16, jnp.full((_LANES,), d, jnp.int32)],
                            vals)
                    return c3
                lax.fori_loop(0, _D // 8, dstep, 0)
                return c2
            lax.fori_loop(0, _G // _LANES, sel_group, 0)

            base8 = (wid * num_granules + g) * (_G // 8)
            pltpu.sync_copy(sel_v, out3.at[pl.ds(base8, _G // 8)])
            return carry

        lax.fori_loop(0, num_granules, step, 0)

    return body


def kernel(token_ids, weight):
    batch, seq = token_ids.shape
    vocab, dim = weight.shape
    total = batch * seq
    num_granules = total // (_NW * _G)

    t2_flat = _build_transpose()(weight.T)
    t2 = t2_flat.reshape(vocab // 2, 2 * dim)

    ids = token_ids.reshape(_NW, num_granules, _G).astype(jnp.int32)
    out = _build_gather(num_granules, total)(ids, t2)
    return out.reshape(batch, seq, dim)


# speedup vs baseline: 3.6688x; 3.6688x over previous
import functools

import jax
import jax.numpy as jnp
from jax import lax
from jax.experimental import pallas as pl
from jax.experimental.pallas import tpu as pltpu
from jax.experimental.pallas import tpu_sc as plsc

_NC = 2
_NS = 16
_NW = _NC * _NS
_G = 128
_D = 64
_LANES = 16

_V = 1000000
_TC = 2048
_TBLOCKS = -(-_V // _TC)


def _tc_transpose_block(x_ref, o_ref):
    xt = x_ref[...].T
    o_ref[...] = jnp.concatenate([xt, xt], axis=1)


def _build_tc_transpose():
    return pl.pallas_call(
        _tc_transpose_block,
        grid=(_TBLOCKS,),
        in_specs=[pl.BlockSpec((_D, _TC), lambda i: (0, i))],
        out_specs=pl.BlockSpec((_TC, 2 * _D), lambda i: (i, 0)),
        out_shape=jax.ShapeDtypeStruct((_V, 2 * _D), jnp.float32),
    )


def _build_gather(num_granules, total_tokens):
    mesh = plsc.VectorSubcoreMesh(core_axis_name="c", subcore_axis_name="s")

    @functools.partial(
        pl.kernel,
        mesh=mesh,
        compiler_params=pltpu.CompilerParams(needs_layout_passes=False),
        out_type=jax.ShapeDtypeStruct((total_tokens, _D), jnp.float32),
        scratch_types=[
            pltpu.VMEM((num_granules, _G), jnp.int32),
            pltpu.VMEM((2, _G, 128), jnp.float32),
            pltpu.VMEM((2, _G // 8, 8, _D), jnp.float32),
            pltpu.SemaphoreType.DMA((2,)),
            pltpu.SemaphoreType.DMA((2,)),
        ],
    )
    def body(ids_hbm, t2_hbm, out_hbm, idx_v, rows_v, sel_v, gsem, wsem):
        wid = lax.axis_index("s") * _NC + lax.axis_index("c")
        pltpu.sync_copy(ids_hbm.at[wid], idx_v)
        out3 = out_hbm.reshape(total_tokens // 8, 8, _D)

        def start_gather(g, p):
            pltpu.async_copy(t2_hbm.at[idx_v.at[g]], rows_v.at[p],
                             gsem.at[p])

        def wait_gather(g, p):
            pltpu.make_async_copy(t2_hbm.at[idx_v.at[g]], rows_v.at[p],
                                  gsem.at[p]).wait()

        def wait_write(p):
            pltpu.make_async_copy(sel_v.at[p], out3.at[pl.ds(0, _G // 8)],
                                  wsem.at[p]).wait()

        start_gather(0, 0)

        def step(g, carry):
            p = g & 1
            @pl.when(g + 1 < num_granules)
            def _():
                start_gather(g + 1, 1 - p)
            wait_gather(g, p)
            @pl.when(g >= 2)
            def _():
                wait_write(p)

            def sel_group(j, c2):
                for k in range(_LANES):
                    r = j * _LANES + k
                    for c in range(4):
                        vals = rows_v[p, r, pl.ds(c * _LANES, _LANES)]
                        sel_v[p, r // 8, r % 8,
                              pl.ds(c * _LANES, _LANES)] = vals
                return c2
            lax.fori_loop(0, _G // _LANES, sel_group, 0)

            base8 = (wid * num_granules + g) * (_G // 8)
            pltpu.async_copy(sel_v.at[p], out3.at[pl.ds(base8, _G // 8)],
                             wsem.at[p])
            return carry

        lax.fori_loop(0, num_granules, step, 0)
        wait_write(num_granules & 1)
        wait_write(1 - (num_granules & 1))

    return body


def kernel(token_ids, weight):
    batch, seq = token_ids.shape
    vocab, dim = weight.shape
    total = batch * seq
    num_granules = total // (_NW * _G)

    t2 = _build_tc_transpose()(weight.T)
    ids = token_ids.reshape(_NW, num_granules, _G).astype(jnp.int32)
    out = _build_gather(num_granules, total)(ids, t2)
    return out.reshape(batch, seq, dim)


# speedup vs baseline: 3.9402x; 1.0740x over previous
import functools

import jax
import jax.numpy as jnp
from jax import lax
from jax.experimental import pallas as pl
from jax.experimental.pallas import tpu as pltpu
from jax.experimental.pallas import tpu_sc as plsc

_NC = 2
_NS = 16
_NW = _NC * _NS
_G = 128
_D = 64
_LANES = 16

_V = 1000000
_TC = 2048
_TBLOCKS = -(-_V // _TC)


def _tc_transpose_block(x_ref, o_ref):
    xt = x_ref[...].T
    o_ref[...] = jnp.concatenate([xt, xt], axis=1)


def _build_tc_transpose():
    return pl.pallas_call(
        _tc_transpose_block,
        grid=(_TBLOCKS,),
        in_specs=[pl.BlockSpec((_D, _TC), lambda i: (0, i))],
        out_specs=pl.BlockSpec((_TC, 2 * _D), lambda i: (i, 0)),
        out_shape=jax.ShapeDtypeStruct((_V, 2 * _D), jnp.float32),
    )


def _build_gather(num_granules, total_tokens):
    mesh = plsc.VectorSubcoreMesh(core_axis_name="c", subcore_axis_name="s")

    @functools.partial(
        pl.kernel,
        mesh=mesh,
        compiler_params=pltpu.CompilerParams(needs_layout_passes=False),
        out_type=jax.ShapeDtypeStruct((total_tokens, _D), jnp.float32),
        scratch_types=[
            pltpu.VMEM((num_granules, _G), jnp.int32),
            pltpu.VMEM((2, _G, 128), jnp.float32),
            pltpu.VMEM((2, _G // 8, 8, _D), jnp.float32),
            pltpu.SemaphoreType.DMA((2,)),
            pltpu.SemaphoreType.DMA((2,)),
        ],
    )
    def body(ids_hbm, t2_hbm, out_hbm, idx_v, rows_v, sel_v, gsem, wsem):
        wid = lax.axis_index("s") * _NC + lax.axis_index("c")
        pltpu.sync_copy(ids_hbm.at[wid], idx_v)
        out3 = out_hbm.reshape(total_tokens // 8, 8, _D)

        def start_gather(g, p):
            pltpu.async_copy(t2_hbm.at[idx_v.at[g]], rows_v.at[p],
                             gsem.at[p])

        def wait_gather(g, p):
            pltpu.make_async_copy(t2_hbm.at[idx_v.at[g]], rows_v.at[p],
                                  gsem.at[p]).wait()

        def wait_write(p):
            pltpu.make_async_copy(sel_v.at[p], out3.at[pl.ds(0, _G // 8)],
                                  wsem.at[p]).wait()

        start_gather(0, 0)

        def step(g, carry):
            p = g & 1
            @pl.when(g + 1 < num_granules)
            def _():
                start_gather(g + 1, 1 - p)
            wait_gather(g, p)
            @pl.when(g >= 2)
            def _():
                wait_write(p)

            for j in range(_G // _LANES):
                for k in range(_LANES):
                    r = j * _LANES + k
                    for c in range(4):
                        vals = rows_v[p, r, pl.ds(c * _LANES, _LANES)]
                        sel_v[p, j * 2 + k // 8, k % 8,
                              pl.ds(c * _LANES, _LANES)] = vals

            base8 = (wid * num_granules + g) * (_G // 8)
            pltpu.async_copy(sel_v.at[p], out3.at[pl.ds(base8, _G // 8)],
                             wsem.at[p])
            return carry

        lax.fori_loop(0, num_granules, step, 0)
        wait_write(num_granules & 1)
        wait_write(1 - (num_granules & 1))

    return body


def kernel(token_ids, weight):
    batch, seq = token_ids.shape
    vocab, dim = weight.shape
    total = batch * seq
    num_granules = total // (_NW * _G)

    t2 = _build_tc_transpose()(weight.T)
    ids = token_ids.reshape(_NW, num_granules, _G).astype(jnp.int32)
    out = _build_gather(num_granules, total)(ids, t2)
    return out.reshape(batch, seq, dim)


# speedup vs baseline: 4.8370x; 1.2276x over previous
import functools

import jax
import jax.numpy as jnp
from jax import lax
from jax.experimental import pallas as pl
from jax.experimental.pallas import tpu as pltpu
from jax.experimental.pallas import tpu_sc as plsc

_NC = 2
_NS = 16
_NW = _NC * _NS
_G = 128
_D = 64
_LANES = 16

_V = 1000000
_TC = 2048
_TBLOCKS = -(-_V // _TC)


def _tc_transpose_block(x_ref, o_ref):
    eye = (lax.broadcasted_iota(jnp.int32, (_D, _D), 0)
           == lax.broadcasted_iota(jnp.int32, (_D, _D), 1)
           ).astype(jnp.float32)
    dn = (((0,), (0,)), ((), ()))
    xt = lax.dot_general(x_ref[...], eye, dn,
                         preferred_element_type=jnp.float32)
    o_ref[...] = jnp.concatenate([xt, xt], axis=1)


def _build_tc_transpose():
    return pl.pallas_call(
        _tc_transpose_block,
        grid=(_TBLOCKS,),
        in_specs=[pl.BlockSpec((_D, _TC), lambda i: (0, i))],
        out_specs=pl.BlockSpec((_TC, 2 * _D), lambda i: (i, 0)),
        out_shape=jax.ShapeDtypeStruct((_V, 2 * _D), jnp.float32),
    )


def _build_gather(num_granules, total_tokens):
    mesh = plsc.VectorSubcoreMesh(core_axis_name="c", subcore_axis_name="s")

    @functools.partial(
        pl.kernel,
        mesh=mesh,
        compiler_params=pltpu.CompilerParams(needs_layout_passes=False),
        out_type=jax.ShapeDtypeStruct((total_tokens, _D), jnp.float32),
        scratch_types=[
            pltpu.VMEM((num_granules, _G), jnp.int32),
            pltpu.VMEM((2, _G, 128), jnp.float32),
            pltpu.VMEM((2, _G // 8, 8, _D), jnp.float32),
            pltpu.SemaphoreType.DMA((2,)),
            pltpu.SemaphoreType.DMA((2,)),
        ],
    )
    def body(ids_hbm, t2_hbm, out_hbm, idx_v, rows_v, sel_v, gsem, wsem):
        wid = lax.axis_index("s") * _NC + lax.axis_index("c")
        pltpu.sync_copy(ids_hbm.at[wid], idx_v)
        out3 = out_hbm.reshape(total_tokens // 8, 8, _D)

        def start_gather(g, p):
            pltpu.async_copy(t2_hbm.at[idx_v.at[g]], rows_v.at[p],
                             gsem.at[p])

        def wait_gather(g, p):
            pltpu.make_async_copy(t2_hbm.at[idx_v.at[g]], rows_v.at[p],
                                  gsem.at[p]).wait()

        def wait_write(p):
            pltpu.make_async_copy(sel_v.at[p], out3.at[pl.ds(0, _G // 8)],
                                  wsem.at[p]).wait()

        start_gather(0, 0)

        def step(g, carry):
            p = g & 1
            @pl.when(g + 1 < num_granules)
            def _():
                start_gather(g + 1, 1 - p)
            wait_gather(g, p)
            @pl.when(g >= 2)
            def _():
                wait_write(p)

            for j in range(_G // _LANES):
                for k in range(_LANES):
                    r = j * _LANES + k
                    for c in range(4):
                        vals = rows_v[p, r, pl.ds(c * _LANES, _LANES)]
                        sel_v[p, j * 2 + k // 8, k % 8,
                              pl.ds(c * _LANES, _LANES)] = vals

            base8 = (wid * num_granules + g) * (_G // 8)
            pltpu.async_copy(sel_v.at[p], out3.at[pl.ds(base8, _G // 8)],
                             wsem.at[p])
            return carry

        lax.fori_loop(0, num_granules, step, 0)
        wait_write(num_granules & 1)
        wait_write(1 - (num_granules & 1))

    return body


def kernel(token_ids, weight):
    batch, seq = token_ids.shape
    vocab, dim = weight.shape
    total = batch * seq
    num_granules = total // (_NW * _G)

    t2 = _build_tc_transpose()(weight.T)
    ids = token_ids.reshape(_NW, num_granules, _G).astype(jnp.int32)
    out = _build_gather(num_granules, total)(ids, t2)
    return out.reshape(batch, seq, dim)
